# Initial kernel scaffold; baseline (speedup 1.0000x reference)
import jax
import jax.numpy as jnp
from jax.experimental import pallas as pl


def kernel(x, emb, W_h, b_h, W_o, b_o):
    raise NotImplementedError("write your pallas kernel here")


# speedup vs baseline: 2.7924x; 2.7924x over previous
import functools

import jax
import jax.numpy as jnp
from jax import lax
from jax.experimental import pallas as pl
from jax.experimental.pallas import tpu as pltpu
from jax.experimental.pallas import tpu_sc as plsc

VOCAB = 1000000
EMB = 32
HID = 64
OUT = 32
B = 16384
L = 200


_TBLK = 8000


def _table_body(emb_ref, wh_ref, bh_ref, wo_ref, bo_ref, out_ref):
    e = emb_ref[...]
    h = jnp.maximum(
        jnp.dot(e, wh_ref[...], preferred_element_type=jnp.float32)
        + bh_ref[...], 0.0)
    out_ref[...] = (
        jnp.dot(h, wo_ref[...], preferred_element_type=jnp.float32)
        + bo_ref[...])


def _compute_table(emb, W_h, b_h, W_o, b_o):
    grid = (VOCAB // _TBLK,)
    return pl.pallas_call(
        _table_body,
        grid=grid,
        in_specs=[
            pl.BlockSpec((_TBLK, EMB), lambda i: (i, 0)),
            pl.BlockSpec((EMB, HID), lambda i: (0, 0)),
            pl.BlockSpec((1, HID), lambda i: (0, 0)),
            pl.BlockSpec((HID, OUT), lambda i: (0, 0)),
            pl.BlockSpec((1, OUT), lambda i: (0, 0)),
        ],
        out_specs=pl.BlockSpec((_TBLK, OUT), lambda i: (i, 0)),
        out_shape=jax.ShapeDtypeStruct((VOCAB, OUT), jnp.float32),
    )(emb, W_h, b_h.reshape(1, HID), W_o, b_o.reshape(1, OUT))


_N_IDX = B * L
_SUB = 128
_NSUB = 16
_CHUNK = _SUB * _NSUB


@functools.lru_cache(maxsize=1)
def _make_gather():
    info = plsc.get_sparse_core_info()
    nc, ns = info.num_cores, info.num_subcores
    nw = nc * ns
    per_w = _N_IDX // nw
    iters = per_w // _CHUNK
    rows_per_iter_2d = _CHUNK // _SUB

    mesh = plsc.VectorSubcoreMesh(core_axis_name="c", subcore_axis_name="s")

    @functools.partial(
        pl.kernel,
        mesh=mesh,
        compiler_params=pltpu.CompilerParams(use_tc_tiling_on_sc=False),
        out_type=jax.ShapeDtypeStruct((_N_IDX, OUT), jnp.float32),
        scratch_types=[
            pltpu.VMEM((_NSUB, _SUB), jnp.int32),
            pltpu.VMEM((_CHUNK, OUT), jnp.float32),
            pltpu.SemaphoreType.DMA,
        ],
    )
    def gathr(table_hbm, idx_hbm, out_hbm, idx_v, rows_v, sem):
        wid = lax.axis_index("s") * nc + lax.axis_index("c")

        def body(g, _):
            r0 = wid * (per_w // _SUB) + g * rows_per_iter_2d
            pltpu.sync_copy(idx_hbm.at[pl.ds(r0, rows_per_iter_2d)], idx_v)
            handles = []
            for j in range(_NSUB):
                handles.append(pltpu.async_copy(
                    table_hbm.at[idx_v.at[j]],
                    rows_v.at[pl.ds(j * _SUB, _SUB)],
                    sem))
            for h in handles:
                h.wait()
            base = wid * per_w + g * _CHUNK
            pltpu.sync_copy(rows_v, out_hbm.at[pl.ds(base, _CHUNK)])
            return 0

        lax.fori_loop(0, iters, body, 0)

    return gathr


_SBLK = 128


def _softmax_body(z_ref, out_ref):
    z = z_ref[...]
    m = jnp.max(z, axis=1, keepdims=True)
    e = jnp.exp(z - m)
    s = jnp.sum(e, axis=1, keepdims=True)
    out_ref[...] = e / s


def _softmax(logits):
    grid = (B // _SBLK,)
    return pl.pallas_call(
        _softmax_body,
        grid=grid,
        in_specs=[pl.BlockSpec((_SBLK, L, OUT), lambda i: (i, 0, 0))],
        out_specs=pl.BlockSpec((_SBLK, L, OUT), lambda i: (i, 0, 0)),
        out_shape=jax.ShapeDtypeStruct((B, L, OUT), jnp.float32),
    )(logits)


def kernel(x, emb, W_h, b_h, W_o, b_o):
    table = _compute_table(emb, W_h, b_h, W_o, b_o)
    idx2d = x.reshape(_N_IDX // _SUB, _SUB).astype(jnp.int32)
    flat = _make_gather()(table, idx2d)
    return _softmax(flat.reshape(B, L, OUT))
